# Initial kernel scaffold; baseline (speedup 1.0000x reference)
import jax
import jax.numpy as jnp
from jax.experimental import pallas as pl


def kernel(original_vertices, original_faces, simplified_vertices, simplified_faces, face_probabilities):
    raise NotImplementedError("write your pallas kernel here")


# speedup vs baseline: 2.3064x; 2.3064x over previous
import functools

import jax
import jax.numpy as jnp
from jax import lax
from jax.experimental import pallas as pl
from jax.experimental.pallas import tpu as pltpu
from jax.experimental.pallas import tpu_sc as plsc

_NUM_SAMPLES = 4
_EPS = 1e-8
_NW = 32
_L = 16

_NOP = 20480
_NQP = 1536
_NSP = 6144
_NVP = 10240
_MB = 512
_NCH = 512
_FWD_STEPS = _NQP // _MB
_REV_STEPS = _NSP // _MB
_STEPS = _FWD_STEPS + _REV_STEPS


def _sc_body(nv_real, ovf_hbm, svf_hbm, of_hbm, sf_hbm, svi_hbm, cf_hbm,
             ob_hbm, ovg_hbm, qb_hbm, sp_hbm,
             ovv, svv, ofv, sfv, sviv, cfv, obv, ovgv, qbv, spv):
    w = lax.axis_index("s") * 2 + lax.axis_index("c")

    pltpu.sync_copy(ovf_hbm, ovv)
    pltpu.sync_copy(svf_hbm, svv)
    pltpu.sync_copy(of_hbm.at[w], ofv)
    pltpu.sync_copy(sf_hbm.at[w], sfv)
    pltpu.sync_copy(svi_hbm.at[w], sviv)
    pltpu.sync_copy(cf_hbm.at[w], cfv)

    ones = jnp.ones((_L,), jnp.float32)

    def bary_group(face_ref, table, out_ref, g, query):
        s = pl.ds(g * _L, _L)
        v0 = face_ref[0, s]
        v1 = face_ref[1, s]
        v2 = face_ref[2, s]
        coords = []
        for c in range(3):
            gc = (plsc.load_gather(table, [v0 * 3 + c])
                  + plsc.load_gather(table, [v1 * 3 + c])
                  + plsc.load_gather(table, [v2 * 3 + c]))
            coords.append(gc / 3.0)
        nrm = coords[0] * coords[0] + coords[1] * coords[1] + coords[2] * coords[2]
        if query:
            for c in range(3):
                out_ref[c, s] = -2.0 * coords[c]
            out_ref[3, s] = nrm
            out_ref[4, s] = ones
        else:
            for c in range(3):
                out_ref[c, s] = coords[c]
            out_ref[3, s] = ones
            out_ref[4, s] = nrm

    def ob_loop(g, _):
        bary_group(ofv, ovv, obv, g, query=False)
        return _

    def qb_loop(g, _):
        bary_group(sfv, svv, qbv, g, query=True)
        return _

    def sp_loop(g, _):
        s = pl.ds(g * _L, _L)
        i0 = sviv[0, s]
        i1 = sviv[1, s]
        i2 = sviv[2, s]
        ca = cfv[0, s]
        cb = cfv[1, s]
        cc = cfv[2, s]
        coords = []
        for c in range(3):
            sc_ = (ca * plsc.load_gather(svv, [i0 * 3 + c])
                   + cb * plsc.load_gather(svv, [i1 * 3 + c])
                   + cc * plsc.load_gather(svv, [i2 * 3 + c]))
            coords.append(sc_)
        nrm = coords[0] * coords[0] + coords[1] * coords[1] + coords[2] * coords[2]
        for c in range(3):
            sp_ref_row = -2.0 * coords[c]
            spv[c, s] = sp_ref_row
        spv[3, s] = nrm
        spv[4, s] = ones
        return _

    def ovg_loop(g, _):
        s = pl.ds(g * _L, _L)
        rows = w * (_NVP // _NW) + g * _L + lax.iota(jnp.int32, _L)
        rows = jnp.minimum(rows, nv_real - 1)
        coords = [plsc.load_gather(ovv, [rows * 3 + c]) for c in range(3)]
        nrm = coords[0] * coords[0] + coords[1] * coords[1] + coords[2] * coords[2]
        for c in range(3):
            ovgv[c, s] = coords[c]
        ovgv[3, s] = ones
        ovgv[4, s] = nrm
        return _

    lax.fori_loop(0, _NOP // _NW // _L, ob_loop, 0)
    lax.fori_loop(0, _NQP // _NW // _L, qb_loop, 0)
    lax.fori_loop(0, _NSP // _NW // _L, sp_loop, 0)
    lax.fori_loop(0, _NVP // _NW // _L, ovg_loop, 0)

    pltpu.sync_copy(obv, ob_hbm.at[w])
    pltpu.sync_copy(ovgv, ovg_hbm.at[w])
    pltpu.sync_copy(qbv, qb_hbm.at[w])
    pltpu.sync_copy(spv, sp_hbm.at[w])


def _sc_stage(ovf, svf, of3, sf3, svi3, cf3, nv_real, interpret=False):
    f32, i32 = jnp.float32, jnp.int32
    mesh = plsc.VectorSubcoreMesh(core_axis_name="c", subcore_axis_name="s")
    out_type = (
        jax.ShapeDtypeStruct((_NW, 5, _NOP // _NW), f32),
        jax.ShapeDtypeStruct((_NW, 5, _NVP // _NW), f32),
        jax.ShapeDtypeStruct((_NW, 5, _NQP // _NW), f32),
        jax.ShapeDtypeStruct((_NW, 5, _NSP // _NW), f32),
    )
    scratch = [
        pltpu.VMEM(ovf.shape, f32), pltpu.VMEM(svf.shape, f32),
        pltpu.VMEM(of3.shape[1:], i32), pltpu.VMEM(sf3.shape[1:], i32),
        pltpu.VMEM(svi3.shape[1:], i32), pltpu.VMEM(cf3.shape[1:], f32),
        pltpu.VMEM((5, _NOP // _NW), f32), pltpu.VMEM((5, _NVP // _NW), f32),
        pltpu.VMEM((5, _NQP // _NW), f32), pltpu.VMEM((5, _NSP // _NW), f32),
    ]
    fn = pl.kernel(functools.partial(_sc_body, nv_real), out_type,
                   mesh=mesh, scratch_types=scratch,
                   compiler_params=pltpu.CompilerParams(
                       needs_layout_passes=False),
                   interpret=interpret)
    return fn(ovf, svf, of3, sf3, svi3, cf3)


def _tc_body(nreal_q, qr, ob, sr, ov, fpf, fps, out, acc, dacc):
    i = pl.program_id(0)

    @pl.when(i == 0)
    def _init():
        acc[0] = 0.0001 * (nreal_q - jnp.sum(fpf[...]))
        acc[1] = 0.0
        acc[2] = 0.0

    def min_block(q_ref, c_ref, qs, n_chunks):
        q3 = q_ref[pl.ds(qs, _MB), 0:3]
        dacc[...] = jnp.full((_MB, _NCH), jnp.inf, jnp.float32)

        def body(n, _):
            cs = pl.multiple_of(n * _NCH, _NCH)
            cross = lax.dot_general(
                q3, c_ref[0:3, pl.ds(cs, _NCH)],
                dimension_numbers=(((1,), (0,)), ((), ())),
                preferred_element_type=jnp.float32)
            sq = c_ref[4:5, pl.ds(cs, _NCH)] + cross
            dacc[...] = jnp.minimum(dacc[...], sq)
            return 0

        lax.fori_loop(0, n_chunks, body, 0)
        qn = q_ref[pl.ds(qs, _MB), 3]
        m = qn + jnp.min(dacc[...], axis=1)
        return jnp.sqrt(jnp.maximum(m, 0.0))

    @pl.when(i < _FWD_STEPS)
    def _fwd():
        qs = pl.multiple_of(i * _MB, _MB)
        dist = min_block(qr, ob, qs, _NOP // _NCH)
        acc[0] += jnp.sum(fpf[0, pl.ds(qs, _MB)] * dist)

    @pl.when(i >= _FWD_STEPS)
    def _rev():
        qs = pl.multiple_of((i - _FWD_STEPS) * _MB, _MB)
        dist = min_block(sr, ov, qs, _NVP // _NCH)
        acc[1] += jnp.sum(fps[0, pl.ds(qs, _MB)] * dist)
        acc[2] = jnp.maximum(acc[2], jnp.max(dist))

    @pl.when(i == _STEPS - 1)
    def _fin():
        out[0, 0] = acc[0] + acc[1] * 0.1 / (acc[2] + _EPS)


def _nn_loss(qr, ob, sr, ov, fpf, fps, nreal_q, interpret=False):
    full = lambda s: pl.BlockSpec(s, lambda i: (0,) * len(s))
    return pl.pallas_call(
        functools.partial(_tc_body, nreal_q),
        grid=(_STEPS,),
        in_specs=[full((_NQP, 5)), full((5, _NOP)), full((_NSP, 5)),
                  full((5, _NVP)), full((1, _NQP)), full((1, _NSP))],
        out_specs=pl.BlockSpec(memory_space=pltpu.SMEM),
        out_shape=jax.ShapeDtypeStruct((1, 1), jnp.float32),
        scratch_shapes=[pltpu.SMEM((3,), jnp.float32),
                        pltpu.VMEM((_MB, _NCH), jnp.float32)],
        interpret=interpret,
    )(qr, ob, sr, ov, fpf, fps)


def _tiles(x, n_pad):
    n = x.shape[1]
    xp = jnp.pad(x, ((0, 0), (0, n_pad - n)), mode="edge")
    return xp.reshape(3, _NW, n_pad // _NW).transpose(1, 0, 2)


def _untile(x3):
    return x3.transpose(1, 0, 2).reshape(5, -1)


def _untile_rows(x3):
    return x3.transpose(0, 2, 1).reshape(-1, 5)


def kernel(original_vertices, original_faces, simplified_vertices,
           simplified_faces, face_probabilities, interpret=False):
    nf = simplified_faces.shape[0]
    nv = original_vertices.shape[0]
    fp = face_probabilities[:nf]

    skey = jax.random.key(42)
    ka, kb = jax.random.split(skey)
    sqrt_r1 = jnp.sqrt(jax.random.uniform(ka, (nf, _NUM_SAMPLES),
                                          dtype=jnp.float32))
    r2 = jax.random.uniform(kb, (nf, _NUM_SAMPLES), dtype=jnp.float32)
    coef = jnp.stack([1.0 - sqrt_r1, sqrt_r1 * (1.0 - r2),
                      sqrt_r1 * r2]).reshape(3, nf * _NUM_SAMPLES)

    of3 = _tiles(original_faces.T.astype(jnp.int32), _NOP)
    sf_t = simplified_faces.T.astype(jnp.int32)
    sf3 = _tiles(sf_t, _NQP)
    svi3 = _tiles(jnp.repeat(sf_t, _NUM_SAMPLES, axis=1), _NSP)
    cf3 = _tiles(coef, _NSP)

    ob3, ovg3, qb3, sp3 = _sc_stage(
        original_vertices.reshape(-1), simplified_vertices.reshape(-1),
        of3, sf3, svi3, cf3, nv, interpret=interpret)

    fpf = jnp.pad(fp, (0, _NQP - nf))[None, :]
    fps = jnp.pad(jnp.repeat(fp, _NUM_SAMPLES),
                  (0, _NSP - nf * _NUM_SAMPLES))[None, :]

    out = _nn_loss(_untile_rows(qb3), _untile(ob3), _untile_rows(sp3),
                   _untile(ovg3), fpf, fps, float(nf), interpret=interpret)
    return out.reshape(())


# speedup vs baseline: 3.3042x; 1.4326x over previous
import functools

import jax
import jax.numpy as jnp
from jax import lax
from jax.experimental import pallas as pl
from jax.experimental.pallas import tpu as pltpu
from jax.experimental.pallas import tpu_sc as plsc

_NUM_SAMPLES = 4
_EPS = 1e-8
_NW = 32
_L = 16

_NOP = 20480
_NQP = 1536
_NSP = 6144
_NVP = 10240
_MB = 512
_NCH = 2048
_FWD_STEPS = _NQP // _MB
_REV_STEPS = _NSP // _MB
_STEPS = _FWD_STEPS + _REV_STEPS


def _sc_body(nv_real, ovf_hbm, svf_hbm, of_hbm, sf_hbm, svi_hbm, cf_hbm,
             ob_hbm, ovg_hbm, qb_hbm, sp_hbm,
             ovv, svv, ofv, sfv, sviv, cfv, obv, ovgv, qbv, spv):
    w = lax.axis_index("s") * 2 + lax.axis_index("c")

    pltpu.sync_copy(ovf_hbm, ovv)
    pltpu.sync_copy(svf_hbm, svv)
    pltpu.sync_copy(of_hbm.at[w], ofv)
    pltpu.sync_copy(sf_hbm.at[w], sfv)
    pltpu.sync_copy(svi_hbm.at[w], sviv)
    pltpu.sync_copy(cf_hbm.at[w], cfv)

    ones = jnp.ones((_L,), jnp.float32)

    def bary_group(face_ref, table, out_ref, g, query):
        s = pl.ds(g * _L, _L)
        v0 = face_ref[0, s]
        v1 = face_ref[1, s]
        v2 = face_ref[2, s]
        coords = []
        for c in range(3):
            gc = (plsc.load_gather(table, [v0 * 3 + c])
                  + plsc.load_gather(table, [v1 * 3 + c])
                  + plsc.load_gather(table, [v2 * 3 + c]))
            coords.append(gc / 3.0)
        nrm = coords[0] * coords[0] + coords[1] * coords[1] + coords[2] * coords[2]
        if query:
            for c in range(3):
                out_ref[c, s] = -2.0 * coords[c]
            out_ref[3, s] = nrm
            out_ref[4, s] = ones
        else:
            for c in range(3):
                out_ref[c, s] = coords[c]
            out_ref[3, s] = ones
            out_ref[4, s] = nrm

    def ob_loop(g, _):
        bary_group(ofv, ovv, obv, g, query=False)
        return _

    def qb_loop(g, _):
        bary_group(sfv, svv, qbv, g, query=True)
        return _

    def sp_loop(g, _):
        s = pl.ds(g * _L, _L)
        i0 = sviv[0, s]
        i1 = sviv[1, s]
        i2 = sviv[2, s]
        ca = cfv[0, s]
        cb = cfv[1, s]
        cc = cfv[2, s]
        coords = []
        for c in range(3):
            sc_ = (ca * plsc.load_gather(svv, [i0 * 3 + c])
                   + cb * plsc.load_gather(svv, [i1 * 3 + c])
                   + cc * plsc.load_gather(svv, [i2 * 3 + c]))
            coords.append(sc_)
        nrm = coords[0] * coords[0] + coords[1] * coords[1] + coords[2] * coords[2]
        for c in range(3):
            sp_ref_row = -2.0 * coords[c]
            spv[c, s] = sp_ref_row
        spv[3, s] = nrm
        spv[4, s] = ones
        return _

    def ovg_loop(g, _):
        s = pl.ds(g * _L, _L)
        rows = w * (_NVP // _NW) + g * _L + lax.iota(jnp.int32, _L)
        rows = jnp.minimum(rows, nv_real - 1)
        coords = [plsc.load_gather(ovv, [rows * 3 + c]) for c in range(3)]
        nrm = coords[0] * coords[0] + coords[1] * coords[1] + coords[2] * coords[2]
        for c in range(3):
            ovgv[c, s] = coords[c]
        ovgv[3, s] = ones
        ovgv[4, s] = nrm
        return _

    lax.fori_loop(0, _NOP // _NW // _L, ob_loop, 0)
    lax.fori_loop(0, _NQP // _NW // _L, qb_loop, 0)
    lax.fori_loop(0, _NSP // _NW // _L, sp_loop, 0)
    lax.fori_loop(0, _NVP // _NW // _L, ovg_loop, 0)

    pltpu.sync_copy(obv, ob_hbm.at[w])
    pltpu.sync_copy(ovgv, ovg_hbm.at[w])
    pltpu.sync_copy(qbv, qb_hbm.at[w])
    pltpu.sync_copy(spv, sp_hbm.at[w])


def _sc_stage(ovf, svf, of3, sf3, svi3, cf3, nv_real, interpret=False):
    f32, i32 = jnp.float32, jnp.int32
    mesh = plsc.VectorSubcoreMesh(core_axis_name="c", subcore_axis_name="s")
    out_type = (
        jax.ShapeDtypeStruct((_NW, 5, _NOP // _NW), f32),
        jax.ShapeDtypeStruct((_NW, 5, _NVP // _NW), f32),
        jax.ShapeDtypeStruct((_NW, 5, _NQP // _NW), f32),
        jax.ShapeDtypeStruct((_NW, 5, _NSP // _NW), f32),
    )
    scratch = [
        pltpu.VMEM(ovf.shape, f32), pltpu.VMEM(svf.shape, f32),
        pltpu.VMEM(of3.shape[1:], i32), pltpu.VMEM(sf3.shape[1:], i32),
        pltpu.VMEM(svi3.shape[1:], i32), pltpu.VMEM(cf3.shape[1:], f32),
        pltpu.VMEM((5, _NOP // _NW), f32), pltpu.VMEM((5, _NVP // _NW), f32),
        pltpu.VMEM((5, _NQP // _NW), f32), pltpu.VMEM((5, _NSP // _NW), f32),
    ]
    fn = pl.kernel(functools.partial(_sc_body, nv_real), out_type,
                   mesh=mesh, scratch_types=scratch,
                   compiler_params=pltpu.CompilerParams(
                       needs_layout_passes=False),
                   interpret=interpret)
    return fn(ovf, svf, of3, sf3, svi3, cf3)


def _tc_body(nreal_q, qr, ob, sr, ov, fpf, fps, out, acc, dacc):
    i = pl.program_id(0)

    @pl.when(i == 0)
    def _init():
        acc[0] = 0.0001 * (nreal_q - jnp.sum(fpf[...]))
        acc[1] = 0.0
        acc[2] = 0.0

    def min_block(q_ref, c_ref, qs, n_chunks):
        q3 = q_ref[pl.ds(qs, _MB), 0:3]

        for n in range(n_chunks):
            cross = lax.dot_general(
                q3, c_ref[0:3, pl.ds(n * _NCH, _NCH)],
                dimension_numbers=(((1,), (0,)), ((), ())),
                preferred_element_type=jnp.float32)
            sq = c_ref[4:5, pl.ds(n * _NCH, _NCH)] + cross
            dacc[...] = sq if n == 0 else jnp.minimum(dacc[...], sq)

        qn = q_ref[pl.ds(qs, _MB), 3]
        m = qn + jnp.min(dacc[...], axis=1)
        return jnp.sqrt(jnp.maximum(m, 0.0))

    @pl.when(i < _FWD_STEPS)
    def _fwd():
        qs = pl.multiple_of(i * _MB, _MB)
        dist = min_block(qr, ob, qs, _NOP // _NCH)
        acc[0] += jnp.sum(fpf[0, pl.ds(qs, _MB)] * dist)

    @pl.when(i >= _FWD_STEPS)
    def _rev():
        qs = pl.multiple_of((i - _FWD_STEPS) * _MB, _MB)
        dist = min_block(sr, ov, qs, _NVP // _NCH)
        acc[1] += jnp.sum(fps[0, pl.ds(qs, _MB)] * dist)
        acc[2] = jnp.maximum(acc[2], jnp.max(dist))

    @pl.when(i == _STEPS - 1)
    def _fin():
        out[0, 0] = acc[0] + acc[1] * 0.1 / (acc[2] + _EPS)


def _nn_loss(qr, ob, sr, ov, fpf, fps, nreal_q, interpret=False):
    full = lambda s: pl.BlockSpec(s, lambda i: (0,) * len(s))
    return pl.pallas_call(
        functools.partial(_tc_body, nreal_q),
        grid=(_STEPS,),
        in_specs=[full((_NQP, 5)), full((5, _NOP)), full((_NSP, 5)),
                  full((5, _NVP)), full((1, _NQP)), full((1, _NSP))],
        out_specs=pl.BlockSpec(memory_space=pltpu.SMEM),
        out_shape=jax.ShapeDtypeStruct((1, 1), jnp.float32),
        scratch_shapes=[pltpu.SMEM((3,), jnp.float32),
                        pltpu.VMEM((_MB, _NCH), jnp.float32)],
        interpret=interpret,
    )(qr, ob, sr, ov, fpf, fps)


def _tiles(x, n_pad):
    n = x.shape[1]
    xp = jnp.pad(x, ((0, 0), (0, n_pad - n)), mode="edge")
    return xp.reshape(3, _NW, n_pad // _NW).transpose(1, 0, 2)


def _untile(x3):
    return x3.transpose(1, 0, 2).reshape(5, -1)


def _untile_rows(x3):
    return x3.transpose(0, 2, 1).reshape(-1, 5)


def kernel(original_vertices, original_faces, simplified_vertices,
           simplified_faces, face_probabilities, interpret=False):
    nf = simplified_faces.shape[0]
    nv = original_vertices.shape[0]
    fp = face_probabilities[:nf]

    skey = jax.random.key(42)
    ka, kb = jax.random.split(skey)
    sqrt_r1 = jnp.sqrt(jax.random.uniform(ka, (nf, _NUM_SAMPLES),
                                          dtype=jnp.float32))
    r2 = jax.random.uniform(kb, (nf, _NUM_SAMPLES), dtype=jnp.float32)
    coef = jnp.stack([1.0 - sqrt_r1, sqrt_r1 * (1.0 - r2),
                      sqrt_r1 * r2]).reshape(3, nf * _NUM_SAMPLES)

    of3 = _tiles(original_faces.T.astype(jnp.int32), _NOP)
    sf_t = simplified_faces.T.astype(jnp.int32)
    sf3 = _tiles(sf_t, _NQP)
    svi3 = _tiles(jnp.repeat(sf_t, _NUM_SAMPLES, axis=1), _NSP)
    cf3 = _tiles(coef, _NSP)

    ob3, ovg3, qb3, sp3 = _sc_stage(
        original_vertices.reshape(-1), simplified_vertices.reshape(-1),
        of3, sf3, svi3, cf3, nv, interpret=interpret)

    fpf = jnp.pad(fp, (0, _NQP - nf))[None, :]
    fps = jnp.pad(jnp.repeat(fp, _NUM_SAMPLES),
                  (0, _NSP - nf * _NUM_SAMPLES))[None, :]

    out = _nn_loss(_untile_rows(qb3), _untile(ob3), _untile_rows(sp3),
                   _untile(ovg3), fpf, fps, float(nf), interpret=interpret)
    return out.reshape(())
